# Initial kernel scaffold; baseline (speedup 1.0000x reference)
import jax
import jax.numpy as jnp
from jax.experimental import pallas as pl


def kernel(x, g_0, g_1, g_2, target_ids, att_l0, att_r0, att_l1, att_r1, gate_a_w, gate_a_b, gate_b_w, gate_b_b, lin_w, lin_b, in_proj_w, in_proj_b, out_proj_w, out_proj_b):
    raise NotImplementedError("write your pallas kernel here")


# speedup vs baseline: 21.8559x; 21.8559x over previous
import functools

import jax
import jax.numpy as jnp
import numpy as np
from jax import lax
from jax.experimental import pallas as pl
from jax.experimental.pallas import tpu as pltpu
from jax.experimental.pallas import tpu_sc as plsc

_N = 10000
_E = 320000
_D = 128
_H = 8
_NF = 64
_L = 32
_EPS = 0.3

_NTILES = 32
_CH = 80
_EPT = _E // _NTILES
_NCH = _EPT // _CH
_NPAD = 10240
_APAD = 10112
_RPT = _APAD // 16

_mesh = plsc.VectorSubcoreMesh(core_axis_name="c", subcore_axis_name="s")


def _sc_hist_body(d0, d1, d2, out0, out1,
                  deg0, deg1, deg2, didx_all, didx_c, ones_v, zbuf):
    c = lax.axis_index("c")
    s = lax.axis_index("s")
    t = c * 16 + s
    zero16 = jnp.zeros((16,), jnp.float32)
    one16 = jnp.full((16,), 1.0, jnp.float32)
    for k in range(_CH // 16):
        ones_v[pl.ds(k * 16, 16)] = one16
    for k in range(640 // 16):
        zbuf[pl.ds(k * 16, 16)] = zero16
    for deg in (deg0, deg1, deg2):
        pltpu.sync_copy(zbuf, deg.at[pl.ds(pl.multiple_of(s * 640, 8), 640)])
    plsc.subcore_barrier()
    for d1d, deg in ((d0, deg0), (d1, deg1), (d2, deg2)):
        pltpu.sync_copy(d1d.at[pl.ds(pl.multiple_of(t * _EPT, 8), _EPT)],
                        didx_all)

        def chunk(i, carry):
            for k in range(_CH // 16):
                didx_c[pl.ds(k * 16, 16)] = didx_all[pl.ds(i * _CH + k * 16, 16)]
            pltpu.sync_copy(ones_v, deg.at[didx_c], add=True)
            return carry

        lax.fori_loop(0, _NCH, chunk, 0)
        plsc.subcore_barrier()

    @pl.when(c == 0)
    def _():
        for g, deg in enumerate((deg0, deg1, deg2)):
            pltpu.sync_copy(deg.at[pl.ds(pl.multiple_of(s * 640, 8), 640)],
                            out0.at[pl.ds(pl.multiple_of(g * _NPAD + s * 640, 8), 640)])

    @pl.when(c == 1)
    def _():
        for g, deg in enumerate((deg0, deg1, deg2)):
            pltpu.sync_copy(deg.at[pl.ds(pl.multiple_of(s * 640, 8), 640)],
                            out1.at[pl.ds(pl.multiple_of(g * _NPAD + s * 640, 8), 640)])


@functools.partial(
    pl.kernel,
    mesh=_mesh,
    out_type=(
        jax.ShapeDtypeStruct((3 * _NPAD,), jnp.float32),
        jax.ShapeDtypeStruct((3 * _NPAD,), jnp.float32),
    ),
    scratch_types=[
        pltpu.VMEM_SHARED((_NPAD,), jnp.float32),
        pltpu.VMEM_SHARED((_NPAD,), jnp.float32),
        pltpu.VMEM_SHARED((_NPAD,), jnp.float32),
        pltpu.VMEM((_EPT,), jnp.int32),
        pltpu.VMEM((_CH,), jnp.int32),
        pltpu.VMEM((_CH,), jnp.float32),
        pltpu.VMEM((640,), jnp.float32),
    ],
)
def _sc_hist(d0, d1, d2, out0, out1,
             deg0, deg1, deg2, didx_all, didx_c, ones_v, zbuf):
    _sc_hist_body(d0, d1, d2, out0, out1,
                  deg0, deg1, deg2, didx_all, didx_c, ones_v, zbuf)


def _sc_pass_body(s1d, d1d, xw, u, v, out,
                  acc, uvm, vvm, sidx_c, didx_c, rows, cbuf, zbuf, sem):
    c = lax.axis_index("c")
    s = lax.axis_index("s")
    t = c * 16 + s
    pltpu.sync_copy(u, uvm)
    pltpu.sync_copy(v, vvm)
    zero16 = jnp.zeros((16,), jnp.float32)
    for r in range(8):
        for k in range(8):
            zbuf[r, pl.ds(k * 16, 16)] = zero16
    for i in range(_RPT // 8):
        pltpu.sync_copy(zbuf, acc.at[pl.ds(pl.multiple_of(s * _RPT + i * 8, 8), 8)])
    plsc.subcore_barrier()

    def chunk(i, carry):
        base = pl.multiple_of(t * _EPT + i * _CH, 8)
        pltpu.sync_copy(s1d.at[pl.ds(base, _CH)], sidx_c)
        pltpu.sync_copy(d1d.at[pl.ds(base, _CH)], didx_c)
        pltpu.async_copy(xw.at[sidx_c], rows, sem).wait()
        for k in range(_CH // 16):
            sl = sidx_c[pl.ds(k * 16, 16)]
            dl = didx_c[pl.ds(k * 16, 16)]
            uu = plsc.load_gather(uvm, [sl])
            vv = plsc.load_gather(vvm, [dl])
            z = uu + vv
            e = jnp.exp(-2.0 * jnp.abs(z))
            tpos = (1.0 - e) / (1.0 + e)
            tt = jnp.where(z < 0.0, -tpos, tpos)
            cbuf[pl.ds(k * 16, 16)] = tt
        for j in range(_CH):
            cb = plsc.load_gather(cbuf, [jnp.full((16,), j, jnp.int32)])
            for k in range(8):
                rows[j, pl.ds(k * 16, 16)] = rows[j, pl.ds(k * 16, 16)] * cb
        pltpu.sync_copy(rows, acc.at[didx_c], add=True)
        return carry

    lax.fori_loop(0, _NCH, chunk, 0)
    plsc.subcore_barrier()
    pltpu.sync_copy(acc.at[pl.ds(pl.multiple_of(s * _RPT, 8), _RPT)],
                    out.at[c, pl.ds(pl.multiple_of(s * _RPT, 8), _RPT)])


@functools.partial(
    pl.kernel,
    mesh=_mesh,
    compiler_params=pltpu.CompilerParams(needs_layout_passes=False),
    out_type=jax.ShapeDtypeStruct((2, _APAD, _D), jnp.float32),
    scratch_types=[
        pltpu.VMEM_SHARED((_APAD, _D), jnp.float32),
        pltpu.VMEM((_N,), jnp.float32),
        pltpu.VMEM((_N,), jnp.float32),
        pltpu.VMEM((_CH,), jnp.int32),
        pltpu.VMEM((_CH,), jnp.int32),
        pltpu.VMEM((_CH, _D), jnp.float32),
        pltpu.VMEM((_CH,), jnp.float32),
        pltpu.VMEM((8, _D), jnp.float32),
        pltpu.SemaphoreType.DMA,
    ],
)
def _sc_pass(s1d, d1d, xw, u, v, out,
             acc, uvm, vvm, sidx_c, didx_c, rows, cbuf, zbuf, sem):
    _sc_pass_body(s1d, d1d, xw, u, v, out,
                  acc, uvm, vvm, sidx_c, didx_c, rows, cbuf, zbuf, sem)


def _sc_gather_body(h, tidx, out, idxv, rowsv, sem):
    c = lax.axis_index("c")
    s = lax.axis_index("s")
    t = c * 16 + s
    per = (_NF * _L) // _NTILES
    pltpu.sync_copy(tidx.at[pl.ds(pl.multiple_of(t * per, 8), per)], idxv)
    pltpu.async_copy(h.at[idxv], rowsv, sem).wait()
    pltpu.sync_copy(rowsv, out.at[pl.ds(pl.multiple_of(t * per, 8), per)])


@functools.partial(
    pl.kernel,
    mesh=_mesh,
    out_type=jax.ShapeDtypeStruct((_NF * _L, _D), jnp.float32),
    scratch_types=[
        pltpu.VMEM(((_NF * _L) // _NTILES,), jnp.int32),
        pltpu.VMEM(((_NF * _L) // _NTILES, _D), jnp.float32),
        pltpu.SemaphoreType.DMA,
    ],
)
def _sc_gather(h, tidx, out, idxv, rowsv, sem):
    _sc_gather_body(h, tidx, out, idxv, rowsv, sem)


def _tc_prep_body(degp0, degp1, x, alw, arw,
                  dinv, al, ar, xw0, xw1, xw2):
    deg = degp0[...] + degp1[...]
    dv = jnp.where(deg > 0.0, lax.rsqrt(jnp.maximum(deg, 1.0)), 0.0)
    dinv[...] = dv
    xx = x[...]
    al[...] = jnp.dot(xx, alw[...], preferred_element_type=jnp.float32)
    ar[...] = jnp.dot(xx, arw[...], preferred_element_type=jnp.float32)
    xw0[...] = dv[0, :_N].reshape(_N, 1) * xx
    xw1[...] = dv[1, :_N].reshape(_N, 1) * xx
    xw2[...] = dv[2, :_N].reshape(_N, 1) * xx


def _tc_prep(degp0, degp1, x, alw, arw):
    return pl.pallas_call(
        _tc_prep_body,
        out_shape=(
            jax.ShapeDtypeStruct((3, _NPAD), jnp.float32),
            jax.ShapeDtypeStruct((_N, 1), jnp.float32),
            jax.ShapeDtypeStruct((_N, 1), jnp.float32),
            jax.ShapeDtypeStruct((_N, _D), jnp.float32),
            jax.ShapeDtypeStruct((_N, _D), jnp.float32),
            jax.ShapeDtypeStruct((_N, _D), jnp.float32),
        ),
    )(degp0, degp1, x, alw, arw)


def _tc_combine_body(p, x, wcol, alw, arw, h, xwh, u, v):
    hh = jnp.maximum(wcol[...] * (p[0, :_N] + p[1, :_N]) + _EPS * x[...], 0.0)
    h[...] = hh
    xwh[...] = wcol[...] * hh
    u[...] = jnp.dot(hh, alw[...], preferred_element_type=jnp.float32)
    v[...] = jnp.dot(hh, arw[...], preferred_element_type=jnp.float32)


def _tc_combine(p, x, wcol, alw, arw):
    return pl.pallas_call(
        _tc_combine_body,
        out_shape=(
            jax.ShapeDtypeStruct((_N, _D), jnp.float32),
            jax.ShapeDtypeStruct((_N, _D), jnp.float32),
            jax.ShapeDtypeStruct((_N, 1), jnp.float32),
            jax.ShapeDtypeStruct((_N, 1), jnp.float32),
        ),
    )(p, x, wcol, alw, arw)


def _tc_fuse_body(x, s_o, a_o, b_o, wda, bda, wdb, bdb, lw1, lw2, lb, h):
    xx = x[...]
    ga = jax.nn.sigmoid(jnp.dot(xx, wda[...],
                                preferred_element_type=jnp.float32) + bda[...])
    gb = jax.nn.sigmoid(jnp.dot(xx, wdb[...],
                                preferred_element_type=jnp.float32) + bdb[...])
    ga_out = ga * a_o[...] + (1.0 - ga) * s_o[...]
    gb_out = gb * b_o[...] + (1.0 - gb) * s_o[...]
    h[...] = jnp.maximum(
        jnp.dot(ga_out, lw1[...], preferred_element_type=jnp.float32)
        + jnp.dot(gb_out, lw2[...], preferred_element_type=jnp.float32)
        + lb[...], 0.0)


def _tc_fuse(x, s_o, a_o, b_o, wda, bda, wdb, bdb, lw1, lw2, lb):
    return pl.pallas_call(
        _tc_fuse_body,
        out_shape=jax.ShapeDtypeStruct((_N, _D), jnp.float32),
    )(x, s_o, a_o, b_o, wda, bda, wdb, bdb, lw1, lw2, lb)


def _tc_mha_body(tgt, rsel, rexp, rsum, mhead, ehead, pcol,
                 wqt, wkt, wvt, bq, bk, bv, wot, bo, outs, ws):
    t = tgt[...]
    kk = jnp.dot(t, wkt[...], preferred_element_type=jnp.float32) + bk[...]
    vv = jnp.dot(t, wvt[...], preferred_element_type=jnp.float32) + bv[...]
    t0 = jnp.dot(rsel[...], t, preferred_element_type=jnp.float32)
    q0 = jnp.dot(t0, wqt[...], preferred_element_type=jnp.float32) + bq[...]
    q0e = jnp.dot(rexp[...], q0, preferred_element_type=jnp.float32)
    s = jnp.dot(kk * q0e, mhead[...], preferred_element_type=jnp.float32) * 0.25
    cmax = jnp.max(s)
    epx = jnp.exp(s - cmax)
    sums = jnp.dot(rsum[...], epx, preferred_element_type=jnp.float32)
    den = jnp.dot(rexp[...], sums, preferred_element_type=jnp.float32)
    attn = epx / den
    wsum = jnp.dot(attn, jnp.full((_H, 1), 1.0 / _H, jnp.float32),
                   preferred_element_type=jnp.float32)
    ws[...] = jnp.dot(rsum[...], pcol[...] * wsum,
                      preferred_element_type=jnp.float32)
    a128 = jnp.dot(attn, ehead[...], preferred_element_type=jnp.float32)
    o = jnp.dot(rsum[...], a128 * vv, preferred_element_type=jnp.float32)
    outs[...] = jnp.dot(o, wot[...], preferred_element_type=jnp.float32) + bo[...]


def _tc_mha(tgt, rsel, rexp, rsum, mhead, ehead, pcol,
            wqt, wkt, wvt, bq, bk, bv, wot, bo):
    return pl.pallas_call(
        _tc_mha_body,
        out_shape=(
            jax.ShapeDtypeStruct((_NF, _D), jnp.float32),
            jax.ShapeDtypeStruct((_NF, _L), jnp.float32),
        ),
    )(tgt, rsel, rexp, rsum, mhead, ehead, pcol,
      wqt, wkt, wvt, bq, bk, bv, wot, bo)


def kernel(x, g_0, g_1, g_2, target_ids, att_l0, att_r0, att_l1, att_r1,
           gate_a_w, gate_a_b, gate_b_w, gate_b_b, lin_w, lin_b,
           in_proj_w, in_proj_b, out_proj_w, out_proj_b):
    graphs = [(g[0], g[1]) for g in (g_0, g_1, g_2)]

    degp0, degp1 = _sc_hist(graphs[0][1], graphs[1][1], graphs[2][1])
    dinv3, al0, ar0, xw0, xw1, xw2 = _tc_prep(
        degp0.reshape(3, _NPAD), degp1.reshape(3, _NPAD), x, att_l0, att_r0)
    xws = (xw0, xw1, xw2)

    hs = []
    for gi, (s1d, d1d) in enumerate(graphs):
        wcol = dinv3[gi, :_N].reshape(_N, 1)
        p1 = _sc_pass(s1d, d1d, xws[gi], al0[:, 0], ar0[:, 0])
        _, xwh, u1, v1 = _tc_combine(p1, x, wcol, att_l1, att_r1)
        p2 = _sc_pass(s1d, d1d, xwh, u1[:, 0], v1[:, 0])
        h2, _, _, _ = _tc_combine(p2, x, wcol, att_l1, att_r1)
        hs.append(h2)
    s_out, a_out, b_out = hs

    wda = (gate_a_w[:, 0] - gate_a_w[:, 1]).reshape(_D, 1)
    bda = (gate_a_b[0] - gate_a_b[1]).reshape(1, 1)
    wdb = (gate_b_w[:, 0] - gate_b_w[:, 1]).reshape(_D, 1)
    bdb = (gate_b_b[0] - gate_b_b[1]).reshape(1, 1)
    lw1 = lin_w[:_D]
    lw2 = lin_w[_D:]
    lb = lin_b.reshape(1, _D)
    h = _tc_fuse(x, s_out, a_out, b_out, wda, bda, wdb, bdb, lw1, lw2, lb)

    tgt = _sc_gather(h, target_ids.reshape(-1))

    ids_e = np.arange(_NF * _L)
    rsel = jnp.asarray((ids_e[None, :] == (np.arange(_NF) * _L)[:, None])
                       .astype(np.float32))
    rsum = jnp.asarray(((ids_e[None, :] // _L) == np.arange(_NF)[:, None])
                       .astype(np.float32))
    rexp = rsum.T
    mhead = jnp.asarray(((np.arange(_D)[:, None] // (_D // _H))
                         == np.arange(_H)[None, :]).astype(np.float32))
    ehead = mhead.T
    pcol = jnp.asarray(((ids_e[:, None] % _L) == np.arange(_L)[None, :])
                       .astype(np.float32))

    wqt = in_proj_w[:_D].T
    wkt = in_proj_w[_D:2 * _D].T
    wvt = in_proj_w[2 * _D:].T
    bq = in_proj_b[:_D].reshape(1, _D)
    bk = in_proj_b[_D:2 * _D].reshape(1, _D)
    bv = in_proj_b[2 * _D:].reshape(1, _D)
    wot = out_proj_w.T
    bo = out_proj_b.reshape(1, _D)

    outs, ws = _tc_mha(tgt, rsel, rexp, rsum, mhead, ehead, pcol,
                       wqt, wkt, wvt, bq, bk, bv, wot, bo)
    return outs, ws
